# Initial kernel scaffold; baseline (speedup 1.0000x reference)
import jax
import jax.numpy as jnp
from jax.experimental import pallas as pl


def kernel(x, edge_index, W1, b1, W2, b2):
    raise NotImplementedError("write your pallas kernel here")


# speedup vs baseline: 11.4211x; 11.4211x over previous
import functools

import jax
import jax.numpy as jnp
from jax import lax
from jax.experimental import pallas as pl
from jax.experimental.pallas import tpu as pltpu
from jax.experimental.pallas import tpu_sc as plsc

N = 10000
NP = 10008
E = 320000
DF = 128
DH = 128
NC_OUT = 40

NCORE = 2
NSUB = 16
B = 128
ET = E // (NCORE * NSUB)
NB = -(-ET // B)
ETP = NB * B
NDR = 10
STRIPE = N // NDR
RCH = 5
RB = STRIPE // RCH

_mesh = plsc.VectorSubcoreMesh(core_axis_name="c", subcore_axis_name="s")


_FIRE = 4
_NCH = NB // _FIRE
_REM = NB - _NCH * _FIRE


@functools.partial(
    pl.kernel,
    out_type=jax.ShapeDtypeStruct((NCORE, NP, 1), jnp.float32),
    mesh=_mesh,
    scratch_types=[
        pltpu.VMEM_SHARED((NP, 1), jnp.float32),
        pltpu.VMEM((NB, B), jnp.int32),
        pltpu.VMEM((B, 1), jnp.float32),
        pltpu.SemaphoreType.DMA,
    ],
)
def _deg_kernel(eidx_hbm, zcol_hbm, ones_hbm, out_hbm, acc, didx, onesv, sem):
    c = lax.axis_index("c")
    s = lax.axis_index("s")
    pltpu.sync_copy(eidx_hbm.at[1, c, s], didx)
    pltpu.sync_copy(ones_hbm, onesv)

    @pl.when(s < NDR)
    def _():
        pltpu.sync_copy(zcol_hbm, acc.at[pl.ds(s * STRIPE, STRIPE)])

    @pl.when(s == NDR)
    def _():
        pltpu.sync_copy(zcol_hbm.at[pl.ds(0, NP - N)], acc.at[pl.ds(N, NP - N)])

    plsc.subcore_barrier()

    def body(ch, _):
        for k in range(_FIRE):
            j = ch * _FIRE + k
            pltpu.async_copy(onesv, acc.at[didx.at[j]], sem, add=True)
        for k in range(_FIRE):
            j = ch * _FIRE + k
            pltpu.make_async_copy(onesv, acc.at[didx.at[j]], sem).wait()
        return 0

    lax.fori_loop(0, _NCH, body, 0)
    for j in range(NB - _REM, NB):
        pltpu.sync_copy(onesv, acc.at[didx.at[j]], add=True)
    plsc.subcore_barrier()

    @pl.when(s < NDR)
    def _():
        rows = pl.ds(s * STRIPE, STRIPE)
        pltpu.sync_copy(acc.at[rows], out_hbm.at[c].at[rows])


@functools.partial(
    pl.kernel,
    out_type=jax.ShapeDtypeStruct((NCORE, NP, DH), jnp.float32),
    mesh=_mesh,
    scratch_types=[
        pltpu.VMEM_SHARED((NP, DH), jnp.float32),
        pltpu.VMEM((NB, B), jnp.int32),
        pltpu.VMEM((NB, B), jnp.int32),
        pltpu.VMEM((B, DH), jnp.float32),
        pltpu.SemaphoreType.DMA,
    ],
)
def _pass_kernel(g_hbm, eidx_hbm, zrow_hbm, out_hbm, acc, sidx, didx, buf,
                 sem):
    c = lax.axis_index("c")
    s = lax.axis_index("s")
    pltpu.sync_copy(eidx_hbm.at[0, c, s], sidx)
    pltpu.sync_copy(eidx_hbm.at[1, c, s], didx)

    @pl.when(s < NDR)
    def _():
        for k in range(RCH):
            rows = pl.ds(s * STRIPE + k * RB, RB)

            @pl.when(c == 0)
            def _():
                pltpu.sync_copy(g_hbm.at[rows], acc.at[rows])

            @pl.when(c == 1)
            def _():
                pltpu.sync_copy(zrow_hbm.at[pl.ds(0, RB)], acc.at[rows])

    @pl.when(s == NDR)
    def _():
        pltpu.sync_copy(zrow_hbm.at[pl.ds(0, NP - N)], acc.at[pl.ds(N, NP - N)])

    plsc.subcore_barrier()

    def body(j, _):
        pltpu.sync_copy(g_hbm.at[sidx.at[j]], buf)
        pltpu.sync_copy(buf, acc.at[didx.at[j]], add=True)
        return 0

    lax.fori_loop(0, NB, body, 0)
    plsc.subcore_barrier()

    @pl.when(s < NDR)
    def _():
        for k in range(RCH):
            rows = pl.ds(s * STRIPE + k * RB, RB)
            pltpu.sync_copy(acc.at[rows], out_hbm.at[c].at[rows])


def _l1_body(degp_ref, x_ref, w1_ref, g1_ref, dis_ref):
    deg = degp_ref[0, :N, 0] + degp_ref[1, :N, 0] + 1.0
    dis = lax.rsqrt(deg)
    dis_ref[...] = dis
    h = jnp.dot(x_ref[...], w1_ref[...], preferred_element_type=jnp.float32)
    g1_ref[:N] = h * dis[:, None]
    g1_ref[N:] = jnp.zeros((NP - N, DH), jnp.float32)


def _l2_body(s1p_ref, dis_ref, b1_ref, g2_ref):
    s1 = s1p_ref[0, :N] + s1p_ref[1, :N]
    dis = dis_ref[...]
    u = jnp.maximum(s1 * dis[:, None] + b1_ref[...][None, :], 0.0)
    g2_ref[:N] = u * dis[:, None]
    g2_ref[N:] = jnp.zeros((NP - N, DH), jnp.float32)


def _fin_body(s2p_ref, dis_ref, w2_ref, b2_ref, out_ref):
    s2 = (s2p_ref[0, :N] + s2p_ref[1, :N]) * dis_ref[...][:, None]
    out_ref[...] = (
        jnp.dot(s2, w2_ref[...], preferred_element_type=jnp.float32)
        + b2_ref[...][None, :]
    )


def _l1_call(degp, x, W1):
    return pl.pallas_call(
        _l1_body,
        out_shape=[
            jax.ShapeDtypeStruct((NP, DH), jnp.float32),
            jax.ShapeDtypeStruct((N,), jnp.float32),
        ],
    )(degp, x, W1)


def _l2_call(s1p, dis, b1):
    return pl.pallas_call(
        _l2_body,
        out_shape=jax.ShapeDtypeStruct((NP, DH), jnp.float32),
    )(s1p, dis, b1)


def _fin_call(s2p, dis, W2, b2):
    return pl.pallas_call(
        _fin_body,
        out_shape=jax.ShapeDtypeStruct((N, NC_OUT), jnp.float32),
    )(s2p, dis, W2, b2)


def kernel(x, edge_index, W1, b1, W2, b2):
    ei = edge_index.astype(jnp.int32)
    ei4 = ei.reshape(2, NCORE, NSUB, ET)
    ei4 = jnp.pad(ei4, ((0, 0), (0, 0), (0, 0), (0, ETP - ET)),
                  constant_values=N)
    eidx = ei4.reshape(2, NCORE, NSUB, NB, B)
    zcol = jnp.zeros((STRIPE, 1), jnp.float32)
    onescol = jnp.ones((B, 1), jnp.float32)
    zrow = jnp.zeros((RB, DH), jnp.float32)

    degp = _deg_kernel(eidx, zcol, onescol)
    g1, dis = _l1_call(degp, x, W1)
    s1p = _pass_kernel(g1, eidx, zrow)
    g2 = _l2_call(s1p, dis, b1)
    s2p = _pass_kernel(g2, eidx, zrow)
    return _fin_call(s2p, dis, W2, b2)
